# Initial kernel scaffold; baseline (speedup 1.0000x reference)
import jax
import jax.numpy as jnp
from jax.experimental import pallas as pl


def kernel(x, edge_index, edge_pairs, W1l, b1l, W1r, W2l, b2l, W2r, Wm1, bm1, Wm2, bm2, Wm3, bm3):
    raise NotImplementedError("write your pallas kernel here")


# speedup vs baseline: 3.6585x; 3.6585x over previous
import functools

import jax
import jax.numpy as jnp
from jax import lax
from jax.experimental import pallas as pl
from jax.experimental.pallas import tpu as pltpu
from jax.experimental.pallas import tpu_sc as plsc

N = 10000
E = 320000
P = 200000
D = 128
H = 128

NC = 2
NS = 16
NW = NC * NS
CH = 80
L = 16

NPAD = 10240
_ROWS_PER_TILE = NPAD // NS
_DEG_ROWS = NPAD // 128


def _make_agg(with_deg):
  nchunks = E // CH
  per_worker = nchunks // NW
  mesh = plsc.VectorSubcoreMesh(
      core_axis_name="c", subcore_axis_name="s", num_cores=NC, num_subcores=NS)

  out_type = [jax.ShapeDtypeStruct((NC, NPAD, H), jnp.float32)]
  scratch = [
      pltpu.VMEM((CH,), jnp.int32),
      pltpu.VMEM((CH,), jnp.int32),
      pltpu.VMEM((CH, H), jnp.float32),
      pltpu.VMEM_SHARED((NPAD, H), jnp.float32),
      pltpu.SemaphoreType.DMA,
  ]
  if with_deg:
    out_type.append(jax.ShapeDtypeStruct((NC, _DEG_ROWS, 128), jnp.float32))
    scratch += [
        pltpu.VMEM((_DEG_ROWS, 128), jnp.float32),
        pltpu.VMEM((_DEG_ROWS,), jnp.int32),
        pltpu.VMEM_SHARED((_DEG_ROWS, 128), jnp.float32),
    ]

  @functools.partial(
      pl.kernel, out_type=out_type, mesh=mesh, scratch_types=scratch,
      compiler_params=pltpu.CompilerParams(needs_layout_passes=False))
  def agg(table, src, dst, zeros, *refs):
    if with_deg:
      (acc_out, deg_out, src_v, dst_v, rows_v, acc_sh, sem,
       hist_v, iota_v, deg_sh) = refs
    else:
      acc_out, src_v, dst_v, rows_v, acc_sh, sem = refs
    c = lax.axis_index("c")
    s = lax.axis_index("s")
    w = c * NS + s
    r0 = s * _ROWS_PER_TILE
    pltpu.sync_copy(zeros.at[pl.ds(r0, _ROWS_PER_TILE)],
                    acc_sh.at[pl.ds(r0, _ROWS_PER_TILE)])
    if with_deg:
      pltpu.sync_copy(zeros.at[pl.ds(0, _DEG_ROWS)], hist_v)

      @pl.when(s < _DEG_ROWS // 8)
      def _():
        pltpu.sync_copy(zeros.at[pl.ds(s * 8, 8)], deg_sh.at[pl.ds(s * 8, 8)])

      for k in range(_DEG_ROWS // L):
        iota_v[pl.ds(k * L, L)] = lax.iota(jnp.int32, L) + (k * L)
    plsc.subcore_barrier()

    ones_l = jnp.full((L,), 1.0, jnp.float32)

    def body(i, carry):
      base = (w * per_worker + i) * CH
      pltpu.sync_copy(src.at[pl.ds(base, CH)], src_v)
      pltpu.sync_copy(dst.at[pl.ds(base, CH)], dst_v)
      pltpu.async_copy(table.at[src_v], rows_v, sem).wait()
      pltpu.sync_copy(rows_v, acc_sh.at[dst_v], add=True)
      if with_deg:
        for k in range(CH // L):
          idx = dst_v[pl.ds(k * L, L)]
          hi = lax.shift_right_logical(idx, 7)
          lo = jnp.bitwise_and(idx, 127)
          plsc.addupdate_scatter(hist_v, [hi, lo], ones_l)
      return carry

    lax.fori_loop(0, per_worker, body, 0)
    plsc.subcore_barrier()
    pltpu.sync_copy(acc_sh.at[pl.ds(r0, _ROWS_PER_TILE)],
                    acc_out.at[c, pl.ds(r0, _ROWS_PER_TILE)])
    if with_deg:
      pltpu.sync_copy(hist_v, deg_sh.at[iota_v], add=True)
      plsc.subcore_barrier()

      @pl.when(s < _DEG_ROWS // 8)
      def _():
        pltpu.sync_copy(deg_sh.at[pl.ds(s * 8, 8)],
                        deg_out.at[c, pl.ds(s * 8, 8)])

  return agg


_agg_deg = _make_agg(True)
_agg = _make_agg(False)


def _make_pair_gather():
  nchunks = P // CH
  mesh = plsc.VectorSubcoreMesh(
      core_axis_name="c", subcore_axis_name="s", num_cores=NC, num_subcores=NS)

  @functools.partial(
      pl.kernel,
      out_type=[
          jax.ShapeDtypeStruct((P, H), jnp.float32),
          jax.ShapeDtypeStruct((P, H), jnp.float32),
      ],
      mesh=mesh,
      scratch_types=[
          pltpu.VMEM((CH,), jnp.int32),
          pltpu.VMEM((CH, H), jnp.float32),
          pltpu.SemaphoreType.DMA,
      ],
  )
  def pair_gather(z, p0, p1, u, v, idx_v, rows_v, sem):
    c = lax.axis_index("c")
    s = lax.axis_index("s")
    w = c * NS + s
    n_w = (nchunks - w + NW - 1) // NW

    def body(i, carry):
      base = (w + i * NW) * CH
      pltpu.sync_copy(p0.at[pl.ds(base, CH)], idx_v)
      pltpu.async_copy(z.at[idx_v], rows_v, sem).wait()
      pltpu.sync_copy(rows_v, u.at[pl.ds(base, CH)])
      pltpu.sync_copy(p1.at[pl.ds(base, CH)], idx_v)
      pltpu.async_copy(z.at[idx_v], rows_v, sem).wait()
      pltpu.sync_copy(rows_v, v.at[pl.ds(base, CH)])
      return carry

    lax.fori_loop(0, n_w, body, 0)

  return pair_gather


_pair_gather = _make_pair_gather()

_BN = 2000
_BP = 2000


def _post1_body(acc_ref, deg_ref, x_ref, wl_ref, wr_ref, b_ref, z_ref):
  agg = acc_ref[0] + acc_ref[1]
  degc = jnp.maximum(deg_ref[0] + deg_ref[1], 1.0)
  zl = lax.dot_general(agg, wl_ref[...], (((1,), (1,)), ((), ())),
                       preferred_element_type=jnp.float32) / degc
  zr = lax.dot_general(x_ref[...], wr_ref[...], (((1,), (1,)), ((), ())),
                       preferred_element_type=jnp.float32)
  z_ref[...] = jnp.maximum(zl + zr + b_ref[...], 0.0)


def _post1(acc, deg, x, wl, wr, b):
  grid = (N // _BN,)
  return pl.pallas_call(
      _post1_body,
      grid=grid,
      in_specs=[
          pl.BlockSpec((NC, _BN, H), lambda i: (0, i, 0)),
          pl.BlockSpec((NC, _BN, 1), lambda i: (0, i, 0)),
          pl.BlockSpec((_BN, D), lambda i: (i, 0)),
          pl.BlockSpec((H, D), lambda i: (0, 0)),
          pl.BlockSpec((H, D), lambda i: (0, 0)),
          pl.BlockSpec((1, H), lambda i: (0, 0)),
      ],
      out_specs=pl.BlockSpec((_BN, H), lambda i: (i, 0)),
      out_shape=jax.ShapeDtypeStruct((N, H), jnp.float32),
      compiler_params=pltpu.CompilerParams(
          dimension_semantics=("parallel",)),
  )(acc, deg, x, wl, wr, b)


def _post2_body(acc_ref, deg_ref, z1_ref, wl_ref, wr_ref, b_ref, z_ref):
  agg = acc_ref[0] + acc_ref[1]
  degc = jnp.maximum(deg_ref[0] + deg_ref[1], 1.0)
  zl = lax.dot_general(agg, wl_ref[...], (((1,), (1,)), ((), ())),
                       preferred_element_type=jnp.float32) / degc
  zr = lax.dot_general(z1_ref[...], wr_ref[...], (((1,), (1,)), ((), ())),
                       preferred_element_type=jnp.float32)
  z_ref[...] = zl + zr + b_ref[...]


def _post2(acc, deg, z1, wl, wr, b):
  grid = (N // _BN,)
  return pl.pallas_call(
      _post2_body,
      grid=grid,
      in_specs=[
          pl.BlockSpec((NC, _BN, H), lambda i: (0, i, 0)),
          pl.BlockSpec((NC, _BN, 1), lambda i: (0, i, 0)),
          pl.BlockSpec((_BN, H), lambda i: (i, 0)),
          pl.BlockSpec((H, H), lambda i: (0, 0)),
          pl.BlockSpec((H, H), lambda i: (0, 0)),
          pl.BlockSpec((1, H), lambda i: (0, 0)),
      ],
      out_specs=pl.BlockSpec((_BN, H), lambda i: (i, 0)),
      out_shape=jax.ShapeDtypeStruct((N, H), jnp.float32),
      compiler_params=pltpu.CompilerParams(
          dimension_semantics=("parallel",)),
  )(acc, deg, z1, wl, wr, b)


def _decode_body(u_ref, v_ref, w1_ref, b1_ref, w2_ref, b2_ref, w3_ref, b3_ref,
                 o_ref):
  h = u_ref[...] * v_ref[...]
  h = jnp.maximum(
      lax.dot_general(h, w1_ref[...], (((1,), (1,)), ((), ())),
                      preferred_element_type=jnp.float32) + b1_ref[...], 0.0)
  h = jnp.maximum(
      lax.dot_general(h, w2_ref[...], (((1,), (1,)), ((), ())),
                      preferred_element_type=jnp.float32) + b2_ref[...], 0.0)
  o = lax.dot_general(h, w3_ref[...], (((1,), (1,)), ((), ())),
                      preferred_element_type=jnp.float32) + b3_ref[0, 0]
  o_ref[...] = o


def _decode(u, v, w1, b1, w2, b2, w3, b3):
  grid = (P // _BP,)
  return pl.pallas_call(
      _decode_body,
      grid=grid,
      in_specs=[
          pl.BlockSpec((_BP, H), lambda i: (i, 0)),
          pl.BlockSpec((_BP, H), lambda i: (i, 0)),
          pl.BlockSpec((H, H), lambda i: (0, 0)),
          pl.BlockSpec((1, H), lambda i: (0, 0)),
          pl.BlockSpec((H // 2, H), lambda i: (0, 0)),
          pl.BlockSpec((1, H // 2), lambda i: (0, 0)),
          pl.BlockSpec((8, H // 2), lambda i: (0, 0)),
          pl.BlockSpec((1, 1), lambda i: (0, 0)),
      ],
      out_specs=pl.BlockSpec((_BP, 8), lambda i: (i, 0)),
      out_shape=jax.ShapeDtypeStruct((P, 8), jnp.float32),
      compiler_params=pltpu.CompilerParams(
          dimension_semantics=("parallel",)),
  )(u, v, w1, b1, w2, b2, w3, b3)


def kernel(x, edge_index, edge_pairs, W1l, b1l, W1r, W2l, b2l, W2r,
           Wm1, bm1, Wm2, bm2, Wm3, bm3):
  src = edge_index[0]
  dst = edge_index[1]
  p0 = edge_pairs[:, 0]
  p1 = edge_pairs[:, 1]

  zeros = jnp.zeros((NPAD, H), jnp.float32)

  acc1, deg = _agg_deg(x, src, dst, zeros)
  deg = deg.reshape(NC, NPAD, 1)
  z1 = _post1(acc1, deg, x, W1l, W1r, b1l.reshape(1, H))

  (acc2,) = _agg(z1, src, dst, zeros)
  z2 = _post2(acc2, deg, z1, W2l, W2r, b2l.reshape(1, H))

  u, v = _pair_gather(z2, p0, p1)
  w3p = jnp.concatenate([Wm3, jnp.zeros((7, H // 2), jnp.float32)], axis=0)
  out = _decode(u, v, Wm1, bm1.reshape(1, H), Wm2, bm2.reshape(1, H // 2),
                w3p, bm3.reshape(1, 1))
  return out[:, 0]
